# baseline (speedup 1.0000x reference)
import functools
import math

import jax
import jax.numpy as jnp
from jax import lax
from jax.experimental import pallas as pl
from jax.experimental.pallas import tpu as pltpu
from jax.experimental.pallas import tpu_sc as plsc

N_MOL = 32
NODE_TILE = 2000
EDGE_TILE = 512
CHUNK = 80
NC = 2
NS = 16
NW = NC * NS
LANES = 16
POSW = 16


def _mesh():
    return plsc.VectorSubcoreMesh(core_axis_name="c", subcore_axis_name="s")


def _sc_geom_build(n_edges, n_pad, hidden):
    epw = n_edges // NW
    nchunks = epw // MCHUNK
    ngroups = nchunks // MIDXG
    npairs = MIDXG // 2
    xcpw = n_pad // (NW * CHUNK)

    def body(pos_hbm, src_hbm, dst_hbm, at_hbm, emb_hbm,
             dv_hbm, x_hbm,
             sidx, didx, aidx, ps0, ps1, pd0, pd1, dv0, dv1, xb,
             gs0, gs1, gd0, gd1, wv0, wv1, xsem):
        c = lax.axis_index("c")
        s = lax.axis_index("s")
        wid = s * NC + c
        row0 = wid * nchunks
        pss, pds, dvs = (ps0, ps1), (pd0, pd1), (dv0, dv1)
        gss, gds, wvs = (gs0, gs1), (gd0, gd1), (wv0, wv1)

        def start_loads(par, jj):
            pltpu.async_copy(pos_hbm.at[sidx.at[jj]], pss[par], gss[par])
            pltpu.async_copy(pos_hbm.at[didx.at[jj]], pds[par], gds[par])

        def wait_loads(par, jj):
            pltpu.make_async_copy(pos_hbm.at[sidx.at[jj]], pss[par],
                                  gss[par]).wait()
            pltpu.make_async_copy(pos_hbm.at[didx.at[jj]], pds[par],
                                  gds[par]).wait()

        def wait_write(par, j):
            pltpu.make_async_copy(
                dvs[par], dv_hbm.at[pl.ds(j * MCHUNK, MCHUNK)],
                wvs[par]).wait()

        for g in range(ngroups):
            pltpu.sync_copy(src_hbm.at[wid, pl.ds(g * MIDXG, MIDXG)], sidx)
            pltpu.sync_copy(dst_hbm.at[wid, pl.ds(g * MIDXG, MIDXG)], didx)
            start_loads(0, 0)
            start_loads(1, 1)

            def pair(p, carry):
                for par in (0, 1):
                    jj = 2 * p + par
                    j = row0 + g * MIDXG + jj
                    wait_loads(par, jj)

                    @pl.when(p > 0)
                    def _():
                        wait_write(par, j - 2)

                    def erow(r, carry3):
                        dvs[par][r] = pds[par][r] - pss[par][r]
                        return carry3

                    lax.fori_loop(0, MCHUNK, erow, 0)
                    pltpu.async_copy(
                        dvs[par], dv_hbm.at[pl.ds(j * MCHUNK, MCHUNK)],
                        wvs[par])

                    @pl.when(p + 1 < npairs)
                    def _():
                        start_loads(par, jj + 2)
                return carry

            lax.fori_loop(0, npairs, pair, 0)
            wait_write(0, row0 + g * MIDXG + 2 * npairs - 2)
            wait_write(1, row0 + g * MIDXG + 2 * npairs - 1)

        pltpu.sync_copy(at_hbm.at[wid], aidx)

        def xchunk(j, carry):
            pltpu.async_copy(emb_hbm.at[aidx.at[j]], xb, xsem).wait()
            pltpu.sync_copy(
                xb, x_hbm.at[pl.ds((wid * xcpw + j) * CHUNK, CHUNK)])
            return carry

        lax.fori_loop(0, xcpw, xchunk, 0)

    return pl.kernel(
        body,
        out_type=[
            jax.ShapeDtypeStruct((n_edges, POSW), jnp.float32),
            jax.ShapeDtypeStruct((n_pad, hidden), jnp.float32),
        ],
        mesh=_mesh(),
        compiler_params=pltpu.CompilerParams(use_tc_tiling_on_sc=False),
        scratch_types=[
            pltpu.VMEM((MIDXG, MCHUNK), jnp.int32),
            pltpu.VMEM((MIDXG, MCHUNK), jnp.int32),
            pltpu.VMEM((xcpw, CHUNK), jnp.int32),
            pltpu.VMEM((MCHUNK, POSW), jnp.float32),
            pltpu.VMEM((MCHUNK, POSW), jnp.float32),
            pltpu.VMEM((MCHUNK, POSW), jnp.float32),
            pltpu.VMEM((MCHUNK, POSW), jnp.float32),
            pltpu.VMEM((MCHUNK, POSW), jnp.float32),
            pltpu.VMEM((MCHUNK, POSW), jnp.float32),
            pltpu.VMEM((CHUNK, hidden), jnp.float32),
            pltpu.SemaphoreType.DMA,
            pltpu.SemaphoreType.DMA,
            pltpu.SemaphoreType.DMA,
            pltpu.SemaphoreType.DMA,
            pltpu.SemaphoreType.DMA,
            pltpu.SemaphoreType.DMA,
            pltpu.SemaphoreType.DMA,
        ],
    )


N_AGG_PAD = 10240
IDXG = 25
MCHUNK = 50
MIDXG = 50


def _sc_msg_build(n_edges, n_nodes, hidden):
    epw = n_edges // NW
    nchunks = epw // MCHUNK
    ngroups = nchunks // MIDXG
    npairs = MIDXG // 2
    rows_per_tile = N_AGG_PAD // NS
    zrows = 40
    zreps = rows_per_tile // zrows

    def body(h_hbm, wf_hbm, src_hbm, dst_hbm,
             agg_hbm,
             sidx, didx, hb0, hb1, wfb0, wfb1, mb0, mb1, agg_sh,
             gs0, gs1, ws0, ws1, ss0, ss1):
        c = lax.axis_index("c")
        s = lax.axis_index("s")
        wid = s * NC + c
        row0 = wid * nchunks
        hbs, wfbs, mbs = (hb0, hb1), (wfb0, wfb1), (mb0, mb1)
        gss, wss, sss = (gs0, gs1), (ws0, ws1), (ss0, ss1)

        zero16 = jnp.zeros((LANES,), jnp.float32)

        def zrow(r, carry):
            for cc in range(hidden // LANES):
                mb0[r, pl.ds(cc * LANES, LANES)] = zero16
            return carry

        lax.fori_loop(0, zrows, zrow, 0)
        for k in range(zreps):
            pltpu.sync_copy(
                mb0.at[pl.ds(0, zrows)],
                agg_sh.at[pl.ds(s * rows_per_tile + k * zrows, zrows)])
        plsc.subcore_barrier()

        def start_loads(par, jj, j):
            pltpu.async_copy(h_hbm.at[sidx.at[jj]], hbs[par], gss[par])
            pltpu.async_copy(
                wf_hbm.at[pl.ds((row0 + j) * MCHUNK, MCHUNK)],
                wfbs[par], wss[par])

        def wait_loads(par, jj, j):
            pltpu.make_async_copy(h_hbm.at[sidx.at[jj]], hbs[par],
                                  gss[par]).wait()
            pltpu.make_async_copy(
                wf_hbm.at[pl.ds((row0 + j) * MCHUNK, MCHUNK)],
                wfbs[par], wss[par]).wait()

        def wait_scat(par):
            pltpu.make_async_copy(mbs[par], agg_sh.at[didx.at[0]],
                                  sss[par]).wait()

        for g in range(ngroups):
            pltpu.sync_copy(src_hbm.at[wid, pl.ds(g * MIDXG, MIDXG)], sidx)
            pltpu.sync_copy(dst_hbm.at[wid, pl.ds(g * MIDXG, MIDXG)], didx)
            start_loads(0, 0, g * MIDXG)
            start_loads(1, 1, g * MIDXG + 1)

            def pair(p, carry):
                for par in (0, 1):
                    jj = 2 * p + par
                    j = g * MIDXG + jj
                    wait_loads(par, jj, j)

                    @pl.when(p > 0)
                    def _():
                        wait_scat(par)

                    def erow(r, carry3):
                        for cc in range(hidden // LANES):
                            sl = pl.ds(cc * LANES, LANES)
                            mbs[par][r, sl] = hbs[par][r, sl] * wfbs[par][r, sl]
                        return carry3

                    lax.fori_loop(0, MCHUNK, erow, 0)
                    pltpu.async_copy(mbs[par], agg_sh.at[didx.at[jj]],
                                     sss[par], add=True)

                    @pl.when(p + 1 < npairs)
                    def _():
                        start_loads(par, jj + 2, j + 2)
                return carry

            lax.fori_loop(0, npairs, pair, 0)
            wait_scat(0)
            wait_scat(1)

        plsc.subcore_barrier()

        for k in range(zreps):
            base = s * rows_per_tile + k * zrows
            pltpu.sync_copy(agg_sh.at[pl.ds(base, zrows)],
                            mb0.at[pl.ds(0, zrows)])
            pltpu.sync_copy(mb0.at[pl.ds(0, zrows)],
                            agg_hbm.at[c, pl.ds(base, zrows)])

    return pl.kernel(
        body,
        out_type=[
            jax.ShapeDtypeStruct((NC, N_AGG_PAD, hidden), jnp.float32),
        ],
        mesh=_mesh(),
        compiler_params=pltpu.CompilerParams(use_tc_tiling_on_sc=False),
        scratch_types=[
            pltpu.VMEM((MIDXG, MCHUNK), jnp.int32),
            pltpu.VMEM((MIDXG, MCHUNK), jnp.int32),
            pltpu.VMEM((MCHUNK, hidden), jnp.float32),
            pltpu.VMEM((MCHUNK, hidden), jnp.float32),
            pltpu.VMEM((MCHUNK, hidden), jnp.float32),
            pltpu.VMEM((MCHUNK, hidden), jnp.float32),
            pltpu.VMEM((MCHUNK, hidden), jnp.float32),
            pltpu.VMEM((MCHUNK, hidden), jnp.float32),
            pltpu.VMEM_SHARED((N_AGG_PAD, hidden), jnp.float32),
            pltpu.SemaphoreType.DMA,
            pltpu.SemaphoreType.DMA,
            pltpu.SemaphoreType.DMA,
            pltpu.SemaphoreType.DMA,
            pltpu.SemaphoreType.DMA,
            pltpu.SemaphoreType.DMA,
        ],
    )


ROWW = 512
GEO2_ROWS = 25


def _geo1_body(dv_ref, d2_ref):
    dv = dv_ref[...]
    d2_ref[...] = jnp.dot(dv * dv, jnp.ones((POSW, 1), jnp.float32),
                          preferred_element_type=jnp.float32)


GEO1_TILE = 2000


def _geo1(dvec):
    e = dvec.shape[0]
    grid = (e // GEO1_TILE,)
    return pl.pallas_call(
        _geo1_body,
        grid=grid,
        in_specs=[
            pl.BlockSpec((GEO1_TILE, POSW), lambda i: (i, 0)),
        ],
        out_specs=pl.BlockSpec((GEO1_TILE, 1), lambda i: (i, 0)),
        out_shape=jax.ShapeDtypeStruct((e, 1), jnp.float32),
    )(dvec)


def _geo2_body(d2_ref, offs_ref, d_ref, cenv_ref):
    d = jnp.sqrt(d2_ref[...] + 1e-12)
    offs = offs_ref[...]
    cutoff = offs[-1:]
    cenv = 0.5 * (jnp.cos(d * (math.pi / cutoff)) + 1.0)
    cenv = jnp.where(d < cutoff, cenv, 0.0)
    d_ref[...] = d
    cenv_ref[...] = cenv


def _geo2(d2r, offs):
    nrows = d2r.shape[0]
    nrbf = offs.shape[0]
    return pl.pallas_call(
        _geo2_body,
        grid=(1,),
        in_specs=[
            pl.BlockSpec((nrows, ROWW), lambda i: (0, 0)),
            pl.BlockSpec((nrbf,), lambda i: (0,)),
        ],
        out_specs=[
            pl.BlockSpec((nrows, ROWW), lambda i: (0, 0)),
            pl.BlockSpec((nrows, ROWW), lambda i: (0, 0)),
        ],
        out_shape=[
            jax.ShapeDtypeStruct((nrows, ROWW), jnp.float32),
            jax.ShapeDtypeStruct((nrows, ROWW), jnp.float32),
        ],
    )(d2r, offs)


def _filt_body(d_ref, cenv_ref, offs_ref, offsc_ref, w1t_ref, b1c_ref,
               w2a_ref, *out_refs):
    n_blocks = len(out_refs)
    offs = offs_ref[...]
    delta = offs[1:2] - offs[0:1]
    coeff = -0.5 / (delta * delta)
    drow = d_ref[...].reshape(1, ROWW)
    cenv = cenv_ref[...].reshape(1, ROWW)
    difft = offsc_ref[...] - drow
    rbft = jnp.exp(coeff * difft * difft)
    for i in range(n_blocks):
        tt = jnp.tanh(jnp.dot(w1t_ref[i], rbft,
                              preferred_element_type=jnp.float32)
                      + b1c_ref[i])
        tt2 = jnp.concatenate([tt * cenv, cenv], axis=0)
        wf = lax.dot_general(tt2, w2a_ref[i],
                             dimension_numbers=(((0,), (0,)), ((), ())),
                             preferred_element_type=jnp.float32)
        out_refs[i][...] = wf


def _filters(d_rows, cenv_rows, offs, offs_col, w1t, b1c, w2a):
    nrows = d_rows.shape[0]
    e = nrows * ROWW
    nb, hidden, nrbf = w1t.shape
    grid = (nrows,)
    d3 = d_rows.reshape(nrows, 1, ROWW)
    c3 = cenv_rows.reshape(nrows, 1, ROWW)
    out = jax.ShapeDtypeStruct((e, hidden), jnp.float32)
    return pl.pallas_call(
        _filt_body,
        grid=grid,
        in_specs=[
            pl.BlockSpec((1, 1, ROWW), lambda i: (i, 0, 0)),
            pl.BlockSpec((1, 1, ROWW), lambda i: (i, 0, 0)),
            pl.BlockSpec((nrbf,), lambda i: (0,)),
            pl.BlockSpec((nrbf, 1), lambda i: (0, 0)),
            pl.BlockSpec((nb, hidden, nrbf), lambda i: (0, 0, 0)),
            pl.BlockSpec((nb, hidden, 1), lambda i: (0, 0, 0)),
            pl.BlockSpec((nb, hidden + 1, hidden), lambda i: (0, 0, 0)),
        ],
        out_specs=[pl.BlockSpec((ROWW, hidden), lambda i: (i, 0))] * nb,
        out_shape=[out] * nb,
    )(d3, c3, offs, offs_col, w1t, b1c, w2a)


def _mm_body(x_ref, w_ref, out_ref):
    out_ref[...] = jnp.dot(x_ref[...], w_ref[...],
                           preferred_element_type=jnp.float32)


def _matmul(x, w):
    n, hidden = x.shape
    grid = (n // NODE_TILE,)
    return pl.pallas_call(
        _mm_body,
        grid=grid,
        in_specs=[
            pl.BlockSpec((NODE_TILE, hidden), lambda i: (i, 0)),
            pl.BlockSpec((hidden, w.shape[1]), lambda i: (0, 0)),
        ],
        out_specs=pl.BlockSpec((NODE_TILE, w.shape[1]), lambda i: (i, 0)),
        out_shape=jax.ShapeDtypeStruct((n, w.shape[1]), jnp.float32),
    )(x, w)


def _upd_body(agg_ref, x_ref, w2_ref, b2_ref, wl_ref, bl_ref, w1n_ref,
              xn_ref, hn_ref):
    a = agg_ref[0] + agg_ref[1]
    t1 = jnp.tanh(jnp.dot(a, w2_ref[...],
                          preferred_element_type=jnp.float32) + b2_ref[...])
    upd = jnp.dot(t1, wl_ref[...],
                  preferred_element_type=jnp.float32) + bl_ref[...]
    xn = x_ref[...] + upd
    xn_ref[...] = xn
    hn_ref[...] = jnp.dot(xn, w1n_ref[...],
                          preferred_element_type=jnp.float32)


def _update(aggp, x, w2, b2, wl, bl, w1n):
    n, hidden = x.shape
    grid = (n // NODE_TILE,)
    return pl.pallas_call(
        _upd_body,
        grid=grid,
        in_specs=[
            pl.BlockSpec((NC, NODE_TILE, hidden), lambda i: (0, i, 0)),
            pl.BlockSpec((NODE_TILE, hidden), lambda i: (i, 0)),
            pl.BlockSpec((hidden, hidden), lambda i: (0, 0)),
            pl.BlockSpec((hidden,), lambda i: (0,)),
            pl.BlockSpec((hidden, hidden), lambda i: (0, 0)),
            pl.BlockSpec((hidden,), lambda i: (0,)),
            pl.BlockSpec((hidden, hidden), lambda i: (0, 0)),
        ],
        out_specs=[
            pl.BlockSpec((NODE_TILE, hidden), lambda i: (i, 0)),
            pl.BlockSpec((NODE_TILE, hidden), lambda i: (i, 0)),
        ],
        out_shape=[
            jax.ShapeDtypeStruct((n, hidden), jnp.float32),
            jax.ShapeDtypeStruct((n, hidden), jnp.float32),
        ],
    )(aggp, x, w2, b2, wl, bl, w1n)


def _upd_last_body(agg_ref, x_ref, w2_ref, b2_ref, wl_ref, bl_ref,
                   ow1_ref, ob1_ref, ow2_ref, ob2_ref, batch_ref, en_ref):
    a = agg_ref[0] + agg_ref[1]
    t1 = jnp.tanh(jnp.dot(a, w2_ref[...],
                          preferred_element_type=jnp.float32) + b2_ref[...])
    upd = jnp.dot(t1, wl_ref[...],
                  preferred_element_type=jnp.float32) + bl_ref[...]
    xn = x_ref[...] + upd
    hout = jnp.tanh(jnp.dot(xn, ow1_ref[...],
                            preferred_element_type=jnp.float32) + ob1_ref[...])
    e = jnp.dot(hout, ow2_ref[...],
                preferred_element_type=jnp.float32) + ob2_ref[...]
    bt = batch_ref[...].reshape(NODE_TILE)
    mids = lax.broadcasted_iota(jnp.int32, (bt.shape[0], N_MOL), 1)
    mask = (bt[:, None] == mids).astype(jnp.float32)
    contrib = jnp.sum(mask * e, axis=0)

    @pl.when(pl.program_id(0) == 0)
    def _():
        en_ref[...] = jnp.zeros_like(en_ref)

    en_ref[...] += contrib


def _update_last(aggp, x, w2, b2, wl, bl, ow1, ob1, ow2, ob2, batch):
    n, hidden = x.shape
    half = ow1.shape[1]
    grid = (n // NODE_TILE,)
    return pl.pallas_call(
        _upd_last_body,
        grid=grid,
        in_specs=[
            pl.BlockSpec((NC, NODE_TILE, hidden), lambda i: (0, i, 0)),
            pl.BlockSpec((NODE_TILE, hidden), lambda i: (i, 0)),
            pl.BlockSpec((hidden, hidden), lambda i: (0, 0)),
            pl.BlockSpec((hidden,), lambda i: (0,)),
            pl.BlockSpec((hidden, hidden), lambda i: (0, 0)),
            pl.BlockSpec((hidden,), lambda i: (0,)),
            pl.BlockSpec((hidden, half), lambda i: (0, 0)),
            pl.BlockSpec((half,), lambda i: (0,)),
            pl.BlockSpec((half, 1), lambda i: (0, 0)),
            pl.BlockSpec((1,), lambda i: (0,)),
            pl.BlockSpec((1, 1, NODE_TILE), lambda i: (i, 0, 0)),
        ],
        out_specs=pl.BlockSpec((N_MOL,), lambda i: (0,)),
        out_shape=jax.ShapeDtypeStruct((N_MOL,), jnp.float32),
    )(aggp, x, w2, b2, wl, bl, ow1, ob1, ow2, ob2,
      batch.reshape(n // NODE_TILE, 1, NODE_TILE))


def kernel(atom_types, pos, edge_index, batch, emb, rbf_offsets,
           lin1_W, filt_W1, filt_b1, filt_W2, filt_b2,
           lin2_W, lin2_b, lin_W, lin_b,
           out_W1, out_b1, out_W2, out_b2):
    n_edges = edge_index.shape[1]
    n_nodes = pos.shape[0]
    hidden = emb.shape[1]
    n_blocks = lin1_W.shape[0]

    epw = n_edges // NW
    src = edge_index[0].astype(jnp.int32).reshape(NW, epw // CHUNK, CHUNK)
    dst = edge_index[1].astype(jnp.int32).reshape(NW, epw // CHUNK, CHUNK)
    srcm = edge_index[0].astype(jnp.int32).reshape(NW, epw // MCHUNK, MCHUNK)
    dstm = edge_index[1].astype(jnp.int32).reshape(NW, epw // MCHUNK, MCHUNK)
    n_pad = ((n_nodes + NW * CHUNK - 1) // (NW * CHUNK)) * (NW * CHUNK)
    at_pad = jnp.pad(atom_types.astype(jnp.int32), (0, n_pad - n_nodes))
    at_pad = at_pad.reshape(NW, n_pad // (NW * CHUNK), CHUNK)
    pos16 = jnp.pad(pos.astype(jnp.float32), ((0, 0), (0, POSW - 3)))

    dvec, x_pad = _sc_geom_build(n_edges, n_pad, hidden)(
        pos16, srcm, dstm, at_pad, emb)
    x = x_pad[:n_nodes]
    d2 = _geo1(dvec)
    d_rows, cenv_rows = _geo2(d2.reshape(n_edges // ROWW, ROWW), rbf_offsets)

    offs_col = rbf_offsets.reshape(-1, 1)
    w1t = jnp.transpose(filt_W1, (0, 2, 1))
    b1c = filt_b1[:, :, None]
    w2a = jnp.concatenate([filt_W2, filt_b2[:, None, :]], axis=1)
    wfs = list(_filters(d_rows, cenv_rows, rbf_offsets, offs_col,
                        w1t[:-1], b1c[:-1], w2a[:-1]))
    wfs += list(_filters(d_rows, cenv_rows, rbf_offsets, offs_col,
                         w1t[-1:], b1c[-1:], w2a[-1:]))
    h = _matmul(x, lin1_W[0])

    msg_call = _sc_msg_build(n_edges, n_nodes, hidden)
    energy = None
    for i in range(n_blocks):
        (aggp,) = msg_call(h, wfs[i], srcm, dstm)
        if i + 1 < n_blocks:
            x, h = _update(aggp, x, lin2_W[i], lin2_b[i],
                           lin_W[i], lin_b[i], lin1_W[i + 1])
        else:
            energy = _update_last(aggp, x, lin2_W[i], lin2_b[i],
                                  lin_W[i], lin_b[i],
                                  out_W1, out_b1, out_W2, out_b2,
                                  batch.astype(jnp.int32))
    return energy


# speedup vs baseline: 1.1147x; 1.1147x over previous
import functools
import math

import jax
import jax.numpy as jnp
from jax import lax
from jax.experimental import pallas as pl
from jax.experimental.pallas import tpu as pltpu
from jax.experimental.pallas import tpu_sc as plsc

N_MOL = 32
NODE_TILE = 2000
EDGE_TILE = 512
CHUNK = 80
NC = 2
NS = 16
NW = NC * NS
LANES = 16
POSW = 16


def _mesh():
    return plsc.VectorSubcoreMesh(core_axis_name="c", subcore_axis_name="s")


def _sc_geom_build(n_edges, n_pad, hidden):
    epw = n_edges // NW
    nchunks = epw // MCHUNK
    ngroups = nchunks // MIDXG
    npairs = MIDXG // 2
    xcpw = n_pad // (NW * CHUNK)

    def body(pos_hbm, src_hbm, dst_hbm, at_hbm, emb_hbm,
             dv_hbm, x_hbm,
             sidx, didx, aidx, ps0, ps1, pd0, pd1, dv0, dv1, xb,
             gs0, gs1, gd0, gd1, wv0, wv1, xsem):
        c = lax.axis_index("c")
        s = lax.axis_index("s")
        wid = s * NC + c
        row0 = wid * nchunks
        pss, pds, dvs = (ps0, ps1), (pd0, pd1), (dv0, dv1)
        gss, gds, wvs = (gs0, gs1), (gd0, gd1), (wv0, wv1)

        def start_loads(par, jj):
            pltpu.async_copy(pos_hbm.at[sidx.at[jj]], pss[par], gss[par])
            pltpu.async_copy(pos_hbm.at[didx.at[jj]], pds[par], gds[par])

        def wait_loads(par, jj):
            pltpu.make_async_copy(pos_hbm.at[sidx.at[jj]], pss[par],
                                  gss[par]).wait()
            pltpu.make_async_copy(pos_hbm.at[didx.at[jj]], pds[par],
                                  gds[par]).wait()

        def wait_write(par, j):
            pltpu.make_async_copy(
                dvs[par], dv_hbm.at[pl.ds(j * MCHUNK, MCHUNK)],
                wvs[par]).wait()

        for g in range(ngroups):
            pltpu.sync_copy(src_hbm.at[wid, pl.ds(g * MIDXG, MIDXG)], sidx)
            pltpu.sync_copy(dst_hbm.at[wid, pl.ds(g * MIDXG, MIDXG)], didx)
            start_loads(0, 0)
            start_loads(1, 1)

            def pair(p, carry):
                for par in (0, 1):
                    jj = 2 * p + par
                    j = row0 + g * MIDXG + jj
                    wait_loads(par, jj)

                    @pl.when(p > 0)
                    def _():
                        wait_write(par, j - 2)

                    def erow(r, carry3):
                        dvs[par][r] = pds[par][r] - pss[par][r]
                        return carry3

                    lax.fori_loop(0, MCHUNK, erow, 0)
                    pltpu.async_copy(
                        dvs[par], dv_hbm.at[pl.ds(j * MCHUNK, MCHUNK)],
                        wvs[par])

                    @pl.when(p + 1 < npairs)
                    def _():
                        start_loads(par, jj + 2)
                return carry

            lax.fori_loop(0, npairs, pair, 0)
            wait_write(0, row0 + g * MIDXG + 2 * npairs - 2)
            wait_write(1, row0 + g * MIDXG + 2 * npairs - 1)

        pltpu.sync_copy(at_hbm.at[wid], aidx)

        def xchunk(j, carry):
            pltpu.async_copy(emb_hbm.at[aidx.at[j]], xb, xsem).wait()
            pltpu.sync_copy(
                xb, x_hbm.at[pl.ds((wid * xcpw + j) * CHUNK, CHUNK)])
            return carry

        lax.fori_loop(0, xcpw, xchunk, 0)

    return pl.kernel(
        body,
        out_type=[
            jax.ShapeDtypeStruct((n_edges, POSW), jnp.float32),
            jax.ShapeDtypeStruct((n_pad, hidden), jnp.float32),
        ],
        mesh=_mesh(),
        compiler_params=pltpu.CompilerParams(use_tc_tiling_on_sc=False),
        scratch_types=[
            pltpu.VMEM((MIDXG, MCHUNK), jnp.int32),
            pltpu.VMEM((MIDXG, MCHUNK), jnp.int32),
            pltpu.VMEM((xcpw, CHUNK), jnp.int32),
            pltpu.VMEM((MCHUNK, POSW), jnp.float32),
            pltpu.VMEM((MCHUNK, POSW), jnp.float32),
            pltpu.VMEM((MCHUNK, POSW), jnp.float32),
            pltpu.VMEM((MCHUNK, POSW), jnp.float32),
            pltpu.VMEM((MCHUNK, POSW), jnp.float32),
            pltpu.VMEM((MCHUNK, POSW), jnp.float32),
            pltpu.VMEM((CHUNK, hidden), jnp.float32),
            pltpu.SemaphoreType.DMA,
            pltpu.SemaphoreType.DMA,
            pltpu.SemaphoreType.DMA,
            pltpu.SemaphoreType.DMA,
            pltpu.SemaphoreType.DMA,
            pltpu.SemaphoreType.DMA,
            pltpu.SemaphoreType.DMA,
        ],
    )


N_AGG_PAD = 10240
IDXG = 25
MCHUNK = 50
MIDXG = 50


def _sc_msg_build(n_edges, n_nodes, hidden):
    epw = n_edges // NW
    nchunks = epw // MCHUNK
    ngroups = nchunks // MIDXG
    npairs = MIDXG // 2
    rows_per_tile = N_AGG_PAD // NS
    zrows = 40
    zreps = rows_per_tile // zrows

    def body(h_hbm, wf_hbm, src_hbm, dst_hbm,
             agg_hbm,
             sidx, didx, hb0, hb1, wfb0, wfb1, mb0, mb1, agg_sh,
             gs0, gs1, ws0, ws1, ss0, ss1):
        c = lax.axis_index("c")
        s = lax.axis_index("s")
        wid = s * NC + c
        row0 = wid * nchunks
        hbs, wfbs, mbs = (hb0, hb1), (wfb0, wfb1), (mb0, mb1)
        gss, wss, sss = (gs0, gs1), (ws0, ws1), (ss0, ss1)

        zero16 = jnp.zeros((LANES,), jnp.float32)

        def zrow(r, carry):
            for cc in range(hidden // LANES):
                mb0[r, pl.ds(cc * LANES, LANES)] = zero16
            return carry

        lax.fori_loop(0, zrows, zrow, 0)
        for k in range(zreps):
            pltpu.sync_copy(
                mb0.at[pl.ds(0, zrows)],
                agg_sh.at[pl.ds(s * rows_per_tile + k * zrows, zrows)])
        plsc.subcore_barrier()

        def start_loads(par, jj, j):
            pltpu.async_copy(h_hbm.at[sidx.at[jj]], hbs[par], gss[par])
            pltpu.async_copy(
                wf_hbm.at[pl.ds((row0 + j) * MCHUNK, MCHUNK)],
                wfbs[par], wss[par])

        def wait_loads(par, jj, j):
            pltpu.make_async_copy(h_hbm.at[sidx.at[jj]], hbs[par],
                                  gss[par]).wait()
            pltpu.make_async_copy(
                wf_hbm.at[pl.ds((row0 + j) * MCHUNK, MCHUNK)],
                wfbs[par], wss[par]).wait()

        def wait_scat(par):
            pltpu.make_async_copy(mbs[par], agg_sh.at[didx.at[0]],
                                  sss[par]).wait()

        for g in range(ngroups):
            pltpu.sync_copy(src_hbm.at[wid, pl.ds(g * MIDXG, MIDXG)], sidx)
            pltpu.sync_copy(dst_hbm.at[wid, pl.ds(g * MIDXG, MIDXG)], didx)
            start_loads(0, 0, g * MIDXG)
            start_loads(1, 1, g * MIDXG + 1)

            def pair(p, carry):
                for par in (0, 1):
                    jj = 2 * p + par
                    j = g * MIDXG + jj
                    wait_loads(par, jj, j)

                    @pl.when(p > 0)
                    def _():
                        wait_scat(par)

                    def erow(r, carry3):
                        for cc in range(hidden // LANES):
                            sl = pl.ds(cc * LANES, LANES)
                            mbs[par][r, sl] = hbs[par][r, sl] * wfbs[par][r, sl]
                        return carry3

                    lax.fori_loop(0, MCHUNK, erow, 0)
                    pltpu.async_copy(mbs[par], agg_sh.at[didx.at[jj]],
                                     sss[par], add=True)

                    @pl.when(p + 1 < npairs)
                    def _():
                        start_loads(par, jj + 2, j + 2)
                return carry

            lax.fori_loop(0, npairs, pair, 0)
            wait_scat(0)
            wait_scat(1)

        plsc.subcore_barrier()

        for k in range(zreps):
            base = s * rows_per_tile + k * zrows
            pltpu.sync_copy(agg_sh.at[pl.ds(base, zrows)],
                            mb0.at[pl.ds(0, zrows)])
            pltpu.sync_copy(mb0.at[pl.ds(0, zrows)],
                            agg_hbm.at[c, pl.ds(base, zrows)])

    return pl.kernel(
        body,
        out_type=[
            jax.ShapeDtypeStruct((NC, N_AGG_PAD, hidden), jnp.float32),
        ],
        mesh=_mesh(),
        compiler_params=pltpu.CompilerParams(use_tc_tiling_on_sc=False),
        scratch_types=[
            pltpu.VMEM((MIDXG, MCHUNK), jnp.int32),
            pltpu.VMEM((MIDXG, MCHUNK), jnp.int32),
            pltpu.VMEM((MCHUNK, hidden), jnp.float32),
            pltpu.VMEM((MCHUNK, hidden), jnp.float32),
            pltpu.VMEM((MCHUNK, hidden), jnp.float32),
            pltpu.VMEM((MCHUNK, hidden), jnp.float32),
            pltpu.VMEM((MCHUNK, hidden), jnp.float32),
            pltpu.VMEM((MCHUNK, hidden), jnp.float32),
            pltpu.VMEM_SHARED((N_AGG_PAD, hidden), jnp.float32),
            pltpu.SemaphoreType.DMA,
            pltpu.SemaphoreType.DMA,
            pltpu.SemaphoreType.DMA,
            pltpu.SemaphoreType.DMA,
            pltpu.SemaphoreType.DMA,
            pltpu.SemaphoreType.DMA,
        ],
    )


ROWW = 640
GEO2_ROWS = 25


def _geo1_body(dv_ref, d2_ref):
    dv = dv_ref[...]
    d2_ref[...] = jnp.dot(dv * dv, jnp.ones((POSW, 1), jnp.float32),
                          preferred_element_type=jnp.float32)


GEO1_TILE = 2000


def _geo1(dvec):
    e = dvec.shape[0]
    grid = (e // GEO1_TILE,)
    return pl.pallas_call(
        _geo1_body,
        grid=grid,
        in_specs=[
            pl.BlockSpec((GEO1_TILE, POSW), lambda i: (i, 0)),
        ],
        out_specs=pl.BlockSpec((GEO1_TILE, 1), lambda i: (i, 0)),
        out_shape=jax.ShapeDtypeStruct((e, 1), jnp.float32),
    )(dvec)


def _geo2_body(d2_ref, offs_ref, d_ref, cenv_ref):
    d = jnp.sqrt(d2_ref[...] + 1e-12)
    offs = offs_ref[...]
    cutoff = offs[-1:]
    cenv = 0.5 * (jnp.cos(d * (math.pi / cutoff)) + 1.0)
    cenv = jnp.where(d < cutoff, cenv, 0.0)
    d_ref[...] = d
    cenv_ref[...] = cenv


def _geo2(d2r, offs):
    nrows = d2r.shape[0]
    nrbf = offs.shape[0]
    return pl.pallas_call(
        _geo2_body,
        grid=(1,),
        in_specs=[
            pl.BlockSpec((nrows, ROWW), lambda i: (0, 0)),
            pl.BlockSpec((nrbf,), lambda i: (0,)),
        ],
        out_specs=[
            pl.BlockSpec((nrows, ROWW), lambda i: (0, 0)),
            pl.BlockSpec((nrows, ROWW), lambda i: (0, 0)),
        ],
        out_shape=[
            jax.ShapeDtypeStruct((nrows, ROWW), jnp.float32),
            jax.ShapeDtypeStruct((nrows, ROWW), jnp.float32),
        ],
    )(d2r, offs)


def _filt_body(d_ref, cenv_ref, offs_ref, offsc_ref, w1t_ref, b1c_ref,
               w2a_ref, *out_refs):
    n_blocks = len(out_refs)
    offs = offs_ref[...]
    delta = offs[1:2] - offs[0:1]
    coeff = -0.5 / (delta * delta)
    drow = d_ref[...].reshape(1, ROWW)
    cenv = cenv_ref[...].reshape(1, ROWW)
    difft = offsc_ref[...] - drow
    rbft = jnp.exp(coeff * difft * difft)
    for i in range(n_blocks):
        tt = jnp.tanh(jnp.dot(w1t_ref[i], rbft,
                              preferred_element_type=jnp.float32)
                      + b1c_ref[i])
        tt2 = jnp.concatenate([tt * cenv, cenv], axis=0)
        wf = lax.dot_general(tt2, w2a_ref[i],
                             dimension_numbers=(((0,), (0,)), ((), ())),
                             preferred_element_type=jnp.float32)
        out_refs[i][...] = wf


def _filters(d_rows, cenv_rows, offs, offs_col, w1t, b1c, w2a):
    nrows = d_rows.shape[0]
    e = nrows * ROWW
    nb, hidden, nrbf = w1t.shape
    grid = (nrows,)
    d3 = d_rows.reshape(nrows, 1, ROWW)
    c3 = cenv_rows.reshape(nrows, 1, ROWW)
    out = jax.ShapeDtypeStruct((e, hidden), jnp.float32)
    return pl.pallas_call(
        _filt_body,
        grid=grid,
        in_specs=[
            pl.BlockSpec((1, 1, ROWW), lambda i: (i, 0, 0)),
            pl.BlockSpec((1, 1, ROWW), lambda i: (i, 0, 0)),
            pl.BlockSpec((nrbf,), lambda i: (0,)),
            pl.BlockSpec((nrbf, 1), lambda i: (0, 0)),
            pl.BlockSpec((nb, hidden, nrbf), lambda i: (0, 0, 0)),
            pl.BlockSpec((nb, hidden, 1), lambda i: (0, 0, 0)),
            pl.BlockSpec((nb, hidden + 1, hidden), lambda i: (0, 0, 0)),
        ],
        out_specs=[pl.BlockSpec((ROWW, hidden), lambda i: (i, 0))] * nb,
        out_shape=[out] * nb,
    )(d3, c3, offs, offs_col, w1t, b1c, w2a)


def _mm_body(x_ref, w_ref, out_ref):
    out_ref[...] = jnp.dot(x_ref[...], w_ref[...],
                           preferred_element_type=jnp.float32)


def _matmul(x, w):
    n, hidden = x.shape
    grid = (n // NODE_TILE,)
    return pl.pallas_call(
        _mm_body,
        grid=grid,
        in_specs=[
            pl.BlockSpec((NODE_TILE, hidden), lambda i: (i, 0)),
            pl.BlockSpec((hidden, w.shape[1]), lambda i: (0, 0)),
        ],
        out_specs=pl.BlockSpec((NODE_TILE, w.shape[1]), lambda i: (i, 0)),
        out_shape=jax.ShapeDtypeStruct((n, w.shape[1]), jnp.float32),
    )(x, w)


def _upd_body(agg_ref, x_ref, w2_ref, b2_ref, wl_ref, bl_ref, w1n_ref,
              xn_ref, hn_ref):
    a = agg_ref[0] + agg_ref[1]
    t1 = jnp.tanh(jnp.dot(a, w2_ref[...],
                          preferred_element_type=jnp.float32) + b2_ref[...])
    upd = jnp.dot(t1, wl_ref[...],
                  preferred_element_type=jnp.float32) + bl_ref[...]
    xn = x_ref[...] + upd
    xn_ref[...] = xn
    hn_ref[...] = jnp.dot(xn, w1n_ref[...],
                          preferred_element_type=jnp.float32)


def _update(aggp, x, w2, b2, wl, bl, w1n):
    n, hidden = x.shape
    grid = (n // NODE_TILE,)
    return pl.pallas_call(
        _upd_body,
        grid=grid,
        in_specs=[
            pl.BlockSpec((NC, NODE_TILE, hidden), lambda i: (0, i, 0)),
            pl.BlockSpec((NODE_TILE, hidden), lambda i: (i, 0)),
            pl.BlockSpec((hidden, hidden), lambda i: (0, 0)),
            pl.BlockSpec((hidden,), lambda i: (0,)),
            pl.BlockSpec((hidden, hidden), lambda i: (0, 0)),
            pl.BlockSpec((hidden,), lambda i: (0,)),
            pl.BlockSpec((hidden, hidden), lambda i: (0, 0)),
        ],
        out_specs=[
            pl.BlockSpec((NODE_TILE, hidden), lambda i: (i, 0)),
            pl.BlockSpec((NODE_TILE, hidden), lambda i: (i, 0)),
        ],
        out_shape=[
            jax.ShapeDtypeStruct((n, hidden), jnp.float32),
            jax.ShapeDtypeStruct((n, hidden), jnp.float32),
        ],
    )(aggp, x, w2, b2, wl, bl, w1n)


def _upd_last_body(agg_ref, x_ref, w2_ref, b2_ref, wl_ref, bl_ref,
                   ow1_ref, ob1_ref, ow2_ref, ob2_ref, batch_ref, en_ref):
    a = agg_ref[0] + agg_ref[1]
    t1 = jnp.tanh(jnp.dot(a, w2_ref[...],
                          preferred_element_type=jnp.float32) + b2_ref[...])
    upd = jnp.dot(t1, wl_ref[...],
                  preferred_element_type=jnp.float32) + bl_ref[...]
    xn = x_ref[...] + upd
    hout = jnp.tanh(jnp.dot(xn, ow1_ref[...],
                            preferred_element_type=jnp.float32) + ob1_ref[...])
    e = jnp.dot(hout, ow2_ref[...],
                preferred_element_type=jnp.float32) + ob2_ref[...]
    bt = batch_ref[...].reshape(NODE_TILE)
    mids = lax.broadcasted_iota(jnp.int32, (bt.shape[0], N_MOL), 1)
    mask = (bt[:, None] == mids).astype(jnp.float32)
    contrib = jnp.sum(mask * e, axis=0)

    @pl.when(pl.program_id(0) == 0)
    def _():
        en_ref[...] = jnp.zeros_like(en_ref)

    en_ref[...] += contrib


def _update_last(aggp, x, w2, b2, wl, bl, ow1, ob1, ow2, ob2, batch):
    n, hidden = x.shape
    half = ow1.shape[1]
    grid = (n // NODE_TILE,)
    return pl.pallas_call(
        _upd_last_body,
        grid=grid,
        in_specs=[
            pl.BlockSpec((NC, NODE_TILE, hidden), lambda i: (0, i, 0)),
            pl.BlockSpec((NODE_TILE, hidden), lambda i: (i, 0)),
            pl.BlockSpec((hidden, hidden), lambda i: (0, 0)),
            pl.BlockSpec((hidden,), lambda i: (0,)),
            pl.BlockSpec((hidden, hidden), lambda i: (0, 0)),
            pl.BlockSpec((hidden,), lambda i: (0,)),
            pl.BlockSpec((hidden, half), lambda i: (0, 0)),
            pl.BlockSpec((half,), lambda i: (0,)),
            pl.BlockSpec((half, 1), lambda i: (0, 0)),
            pl.BlockSpec((1,), lambda i: (0,)),
            pl.BlockSpec((1, 1, NODE_TILE), lambda i: (i, 0, 0)),
        ],
        out_specs=pl.BlockSpec((N_MOL,), lambda i: (0,)),
        out_shape=jax.ShapeDtypeStruct((N_MOL,), jnp.float32),
    )(aggp, x, w2, b2, wl, bl, ow1, ob1, ow2, ob2,
      batch.reshape(n // NODE_TILE, 1, NODE_TILE))


def kernel(atom_types, pos, edge_index, batch, emb, rbf_offsets,
           lin1_W, filt_W1, filt_b1, filt_W2, filt_b2,
           lin2_W, lin2_b, lin_W, lin_b,
           out_W1, out_b1, out_W2, out_b2):
    n_edges = edge_index.shape[1]
    n_nodes = pos.shape[0]
    hidden = emb.shape[1]
    n_blocks = lin1_W.shape[0]

    epw = n_edges // NW
    src = edge_index[0].astype(jnp.int32).reshape(NW, epw // CHUNK, CHUNK)
    dst = edge_index[1].astype(jnp.int32).reshape(NW, epw // CHUNK, CHUNK)
    srcm = edge_index[0].astype(jnp.int32).reshape(NW, epw // MCHUNK, MCHUNK)
    dstm = edge_index[1].astype(jnp.int32).reshape(NW, epw // MCHUNK, MCHUNK)
    n_pad = ((n_nodes + NW * CHUNK - 1) // (NW * CHUNK)) * (NW * CHUNK)
    at_pad = jnp.pad(atom_types.astype(jnp.int32), (0, n_pad - n_nodes))
    at_pad = at_pad.reshape(NW, n_pad // (NW * CHUNK), CHUNK)
    pos16 = jnp.pad(pos.astype(jnp.float32), ((0, 0), (0, POSW - 3)))

    dvec, x_pad = _sc_geom_build(n_edges, n_pad, hidden)(
        pos16, srcm, dstm, at_pad, emb)
    x = x_pad[:n_nodes]
    d2 = _geo1(dvec)
    d_rows, cenv_rows = _geo2(d2.reshape(n_edges // ROWW, ROWW), rbf_offsets)

    offs_col = rbf_offsets.reshape(-1, 1)
    w1t = jnp.transpose(filt_W1, (0, 2, 1))
    b1c = filt_b1[:, :, None]
    w2a = jnp.concatenate([filt_W2, filt_b2[:, None, :]], axis=1)
    wfs = list(_filters(d_rows, cenv_rows, rbf_offsets, offs_col,
                        w1t, b1c, w2a))
    h = _matmul(x, lin1_W[0])

    msg_call = _sc_msg_build(n_edges, n_nodes, hidden)
    energy = None
    for i in range(n_blocks):
        (aggp,) = msg_call(h, wfs[i], srcm, dstm)
        if i + 1 < n_blocks:
            x, h = _update(aggp, x, lin2_W[i], lin2_b[i],
                           lin_W[i], lin_b[i], lin1_W[i + 1])
        else:
            energy = _update_last(aggp, x, lin2_W[i], lin2_b[i],
                                  lin_W[i], lin_b[i],
                                  out_W1, out_b1, out_W2, out_b2,
                                  batch.astype(jnp.int32))
    return energy
